# baseline (speedup 1.0000x reference)
import functools

import jax
import jax.numpy as jnp
import numpy as np
from jax import lax
from jax.experimental import pallas as pl
from jax.experimental.pallas import tpu as pltpu, tpu_sc as plsc

N_ROWS = 2097152
QROWS = N_ROWS // 128
BLK_Q = 1024

SC_CORES = 2
SC_SUBCORES = 16
SC_LANES = 16
SC_WORKERS = SC_CORES * SC_SUBCORES
SC_B_PER_W = N_ROWS // SC_WORKERS
SC_CHUNK = 16384


@functools.cache
def _sc_gather_fn():
    mesh = plsc.VectorSubcoreMesh(core_axis_name="c", subcore_axis_name="s")

    @functools.partial(
        pl.kernel, mesh=mesh,
        compiler_params=pltpu.CompilerParams(needs_layout_passes=False),
        out_type=jax.ShapeDtypeStruct((N_ROWS,), jnp.float32),
        scratch_types=[
            pltpu.VMEM((1024,), jnp.float32),
            pltpu.VMEM((SC_CHUNK,), jnp.int32),
            pltpu.VMEM((SC_CHUNK,), jnp.float32),
        ],
    )
    def sc_gather(tab_hbm, t_hbm, out_hbm, tab_v, idx_v, val_v):
        wid = lax.axis_index("s") * SC_CORES + lax.axis_index("c")
        pltpu.sync_copy(tab_hbm, tab_v)
        base = wid * SC_B_PER_W
        for c in range(SC_B_PER_W // SC_CHUNK):
            cbase = base + c * SC_CHUNK
            pltpu.sync_copy(t_hbm.at[pl.ds(cbase, SC_CHUNK)], idx_v)

            @pl.loop(0, SC_CHUNK // SC_LANES, unroll=8)
            def body(i):
                idx = idx_v[pl.ds(i * SC_LANES, SC_LANES)]
                val_v[pl.ds(i * SC_LANES, SC_LANES)] = plsc.load_gather(
                    tab_v, [idx])

            pltpu.sync_copy(val_v, out_hbm.at[pl.ds(cbase, SC_CHUNK)])

    return sc_gather


def _threefry_bits(n):
    ks0 = jnp.uint32(0)
    ks1 = jnp.uint32(1)
    ks2 = jnp.uint32(0x1BD11BDA) ^ ks0 ^ ks1
    x0 = jnp.zeros_like(n) + ks0
    x1 = n + ks1

    def rnd(x0, x1, r):
        x0 = x0 + x1
        x1 = (x1 << jnp.uint32(r)) | (x1 >> jnp.uint32(32 - r))
        x1 = x0 ^ x1
        return x0, x1

    r_even = (13, 15, 26, 6)
    r_odd = (17, 29, 16, 24)
    for r in r_even:
        x0, x1 = rnd(x0, x1, r)
    x0 = x0 + ks1
    x1 = x1 + ks2 + jnp.uint32(1)
    for r in r_odd:
        x0, x1 = rnd(x0, x1, r)
    x0 = x0 + ks2
    x1 = x1 + ks0 + jnp.uint32(2)
    for r in r_even:
        x0, x1 = rnd(x0, x1, r)
    x0 = x0 + ks0
    x1 = x1 + ks1 + jnp.uint32(3)
    for r in r_odd:
        x0, x1 = rnd(x0, x1, r)
    x0 = x0 + ks1
    x1 = x1 + ks2 + jnp.uint32(4)
    for r in r_even:
        x0, x1 = rnd(x0, x1, r)
    x0 = x0 + ks2
    x1 = x1 + ks0 + jnp.uint32(5)
    return x0 ^ x1


def _bits_to_normal(bits):
    fb = (bits >> jnp.uint32(9)) | jnp.uint32(0x3F800000)
    floats = jax.lax.bitcast_convert_type(fb, jnp.float32) - jnp.float32(1.0)
    lo = jnp.float32(np.nextafter(np.float32(-1.0), np.float32(0.0)))
    hi = jnp.float32(1.0)
    u = jnp.maximum(lo, floats * (hi - lo) + lo)
    return jnp.float32(np.sqrt(2)) * jax.lax.erf_inv(u)


def _fused_kernel(ab_ref, p0_ref, out_ref, e_ref):
    g = pl.program_id(0)
    ab = ab_ref[...]
    c0 = jnp.sqrt(ab)
    c1 = jnp.sqrt(jnp.maximum(1.0 - ab, 0.0))

    row = jax.lax.broadcasted_iota(jnp.int32, (BLK_Q, 128), 0)
    lane = jax.lax.broadcasted_iota(jnp.int32, (BLK_Q, 128), 1)
    i = (g * BLK_Q + row) * 128 + lane
    n3 = (i * 3).astype(jnp.uint32)
    for j in range(3):
        e = _bits_to_normal(_threefry_bits(n3 + jnp.uint32(j)))
        out_ref[j] = c0 * p0_ref[j] + c1 * e
        e_ref[j] = e


@jax.jit
def _run(abv, p0v):
    grid = (QROWS // BLK_Q,)
    return pl.pallas_call(
        _fused_kernel,
        grid=grid,
        in_specs=[
            pl.BlockSpec((BLK_Q, 128), lambda g: (g, 0)),
            pl.BlockSpec((3, BLK_Q, 128), lambda g: (0, g, 0)),
        ],
        out_specs=[
            pl.BlockSpec((3, BLK_Q, 128), lambda g: (0, g, 0)),
            pl.BlockSpec((3, BLK_Q, 128), lambda g: (0, g, 0)),
        ],
        out_shape=[
            jax.ShapeDtypeStruct((3, QROWS, 128), jnp.float32),
            jax.ShapeDtypeStruct((3, QROWS, 128), jnp.float32),
        ],
    )(abv, p0v)


def kernel(p_0, t, alpha_bars):
    tab = jnp.pad(alpha_bars, (0, 1024 - alpha_bars.shape[0]))
    ab = _sc_gather_fn()(tab, t)
    abv = ab.reshape(QROWS, 128)
    p0v = p_0.T.reshape(3, QROWS, 128)
    out3, e3 = _run(abv, p0v)
    out = out3.reshape(3, N_ROWS).T
    e = e3.reshape(3, N_ROWS).T
    return out, e


# speedup vs baseline: 1.0072x; 1.0072x over previous
import functools

import jax
import jax.numpy as jnp
import numpy as np
from jax import lax
from jax.experimental import pallas as pl
from jax.experimental.pallas import tpu as pltpu, tpu_sc as plsc

N_ROWS = 2097152
QROWS = N_ROWS // 128
BLK_Q = 1024

SC_CORES = 2
SC_SUBCORES = 16
SC_LANES = 16
SC_WORKERS = SC_CORES * SC_SUBCORES
SC_B_PER_W = N_ROWS // SC_WORKERS
SC_CHUNK = 16384


@functools.cache
def _sc_gather_fn():
    mesh = plsc.VectorSubcoreMesh(core_axis_name="c", subcore_axis_name="s")

    @functools.partial(
        pl.kernel, mesh=mesh,
        compiler_params=pltpu.CompilerParams(needs_layout_passes=False),
        out_type=jax.ShapeDtypeStruct((N_ROWS,), jnp.float32),
        scratch_types=[
            pltpu.VMEM((1024,), jnp.float32),
            pltpu.VMEM((2, SC_CHUNK), jnp.int32),
            pltpu.VMEM((2, SC_CHUNK), jnp.float32),
            pltpu.SemaphoreType.DMA,
            pltpu.SemaphoreType.DMA,
            pltpu.SemaphoreType.DMA,
            pltpu.SemaphoreType.DMA,
        ],
    )
    def sc_gather(tab_hbm, t_hbm, out_hbm, tab_v, idx_v, val_v,
                  in_sem0, in_sem1, out_sem0, out_sem1):
        in_sems = (in_sem0, in_sem1)
        out_sems = (out_sem0, out_sem1)
        wid = lax.axis_index("s") * SC_CORES + lax.axis_index("c")
        pltpu.sync_copy(tab_hbm, tab_v)
        base = wid * SC_B_PER_W
        nchunk = SC_B_PER_W // SC_CHUNK
        h_in = {}
        h_out = {}
        h_in[0] = pltpu.async_copy(
            t_hbm.at[pl.ds(base, SC_CHUNK)], idx_v.at[0], in_sems[0])
        for c in range(nchunk):
            b = c % 2
            if c + 1 < nchunk:
                h_in[c + 1] = pltpu.async_copy(
                    t_hbm.at[pl.ds(base + (c + 1) * SC_CHUNK, SC_CHUNK)],
                    idx_v.at[1 - b], in_sems[1 - b])
            h_in[c].wait()
            if c >= 2:
                h_out[c - 2].wait()

            @pl.loop(0, SC_CHUNK // SC_LANES, unroll=8)
            def body(i):
                idx = idx_v[b, pl.ds(i * SC_LANES, SC_LANES)]
                val_v[b, pl.ds(i * SC_LANES, SC_LANES)] = plsc.load_gather(
                    tab_v, [idx])

            h_out[c] = pltpu.async_copy(
                val_v.at[b], out_hbm.at[pl.ds(base + c * SC_CHUNK, SC_CHUNK)],
                out_sems[b])
        for c in range(max(nchunk - 2, 0), nchunk):
            h_out[c].wait()

    return sc_gather


def _threefry_bits(n):
    ks0 = jnp.uint32(0)
    ks1 = jnp.uint32(1)
    ks2 = jnp.uint32(0x1BD11BDA) ^ ks0 ^ ks1
    x0 = jnp.zeros_like(n) + ks0
    x1 = n + ks1

    def rnd(x0, x1, r):
        x0 = x0 + x1
        x1 = (x1 << jnp.uint32(r)) | (x1 >> jnp.uint32(32 - r))
        x1 = x0 ^ x1
        return x0, x1

    r_even = (13, 15, 26, 6)
    r_odd = (17, 29, 16, 24)
    for r in r_even:
        x0, x1 = rnd(x0, x1, r)
    x0 = x0 + ks1
    x1 = x1 + ks2 + jnp.uint32(1)
    for r in r_odd:
        x0, x1 = rnd(x0, x1, r)
    x0 = x0 + ks2
    x1 = x1 + ks0 + jnp.uint32(2)
    for r in r_even:
        x0, x1 = rnd(x0, x1, r)
    x0 = x0 + ks0
    x1 = x1 + ks1 + jnp.uint32(3)
    for r in r_odd:
        x0, x1 = rnd(x0, x1, r)
    x0 = x0 + ks1
    x1 = x1 + ks2 + jnp.uint32(4)
    for r in r_even:
        x0, x1 = rnd(x0, x1, r)
    x0 = x0 + ks2
    x1 = x1 + ks0 + jnp.uint32(5)
    return x0 ^ x1


def _bits_to_normal(bits):
    fb = (bits >> jnp.uint32(9)) | jnp.uint32(0x3F800000)
    floats = jax.lax.bitcast_convert_type(fb, jnp.float32) - jnp.float32(1.0)
    lo = jnp.float32(np.nextafter(np.float32(-1.0), np.float32(0.0)))
    hi = jnp.float32(1.0)
    u = jnp.maximum(lo, floats * (hi - lo) + lo)
    return jnp.float32(np.sqrt(2)) * jax.lax.erf_inv(u)


def _fused_kernel(ab_ref, p0_ref, out_ref, e_ref):
    g = pl.program_id(0)
    ab = ab_ref[...]
    c0 = jnp.sqrt(ab)
    c1 = jnp.sqrt(jnp.maximum(1.0 - ab, 0.0))

    row = jax.lax.broadcasted_iota(jnp.int32, (BLK_Q, 128), 0)
    lane = jax.lax.broadcasted_iota(jnp.int32, (BLK_Q, 128), 1)
    i = (g * BLK_Q + row) * 128 + lane
    n3 = (i * 3).astype(jnp.uint32)
    for j in range(3):
        e = _bits_to_normal(_threefry_bits(n3 + jnp.uint32(j)))
        out_ref[j] = c0 * p0_ref[j] + c1 * e
        e_ref[j] = e


@jax.jit
def _run(abv, p0v):
    grid = (QROWS // BLK_Q,)
    return pl.pallas_call(
        _fused_kernel,
        grid=grid,
        in_specs=[
            pl.BlockSpec((BLK_Q, 128), lambda g: (g, 0)),
            pl.BlockSpec((3, BLK_Q, 128), lambda g: (0, g, 0)),
        ],
        out_specs=[
            pl.BlockSpec((3, BLK_Q, 128), lambda g: (0, g, 0)),
            pl.BlockSpec((3, BLK_Q, 128), lambda g: (0, g, 0)),
        ],
        out_shape=[
            jax.ShapeDtypeStruct((3, QROWS, 128), jnp.float32),
            jax.ShapeDtypeStruct((3, QROWS, 128), jnp.float32),
        ],
    )(abv, p0v)


def kernel(p_0, t, alpha_bars):
    tab = jnp.pad(alpha_bars, (0, 1024 - alpha_bars.shape[0]))
    ab = _sc_gather_fn()(tab, t)
    abv = ab.reshape(QROWS, 128)
    p0v = p_0.T.reshape(3, QROWS, 128)
    out3, e3 = _run(abv, p0v)
    out = out3.reshape(3, N_ROWS).T
    e = e3.reshape(3, N_ROWS).T
    return out, e
